# Initial kernel scaffold; baseline (speedup 1.0000x reference)
import jax
import jax.numpy as jnp
from jax.experimental import pallas as pl


def kernel(x, edge_index, edge, W0, b0, W1, att_src1, att_dst1, bias1, W2, att_src2, att_dst2, bias2):
    raise NotImplementedError("write your pallas kernel here")


# speedup vs baseline: 11.8399x; 11.8399x over previous
import functools

import jax
import jax.numpy as jnp
from jax import lax
from jax.experimental import pallas as pl
from jax.experimental.pallas import tpu as pltpu
from jax.experimental.pallas import tpu_sc as plsc

N = 10000
E = 320000
D = 128
H1 = 256
H2 = 128

NC = 2
NS = 16
L = 16
B = 128

E_MP = E + N
NB_MP = -(-E_MP // (NS * B))
EPT_MP = NB_MP * B
E_MP_PAD = NS * EPT_MP

NB_HD = -(-E // (NC * NS * B))
E_HD_PAD = NC * NS * NB_HD * B

NP = 10240
NPT = NP // NS


def _tc1_body(x_ref, w0_ref, b0_ref, w1_ref, att_ref, ha_ref, hb_ref, al_ref):
    h0 = jnp.dot(x_ref[...], w0_ref[...], preferred_element_type=jnp.float32)
    h0 = jnp.maximum(h0 + b0_ref[...], 0.0)
    h1 = jnp.dot(h0, w1_ref[...], preferred_element_type=jnp.float32)
    ha_ref[...] = h1[:, : H1 // 2]
    hb_ref[...] = h1[:, H1 // 2 :]
    a_src = jnp.sum(h1 * att_ref[0:1, :], axis=1, keepdims=True)
    a_dst = jnp.sum(h1 * att_ref[1:2, :], axis=1, keepdims=True)
    al_ref[...] = jnp.concatenate([a_src, a_dst], axis=1)


def _tc1(x, W0, b0, W1, att1):
    return pl.pallas_call(
        _tc1_body,
        out_shape=[
            jax.ShapeDtypeStruct((N, H1 // 2), jnp.float32),
            jax.ShapeDtypeStruct((N, H1 // 2), jnp.float32),
            jax.ShapeDtypeStruct((N, 2), jnp.float32),
        ],
    )(x, W0, b0, W1, att1)


def _tc2_body(ua_ref, ub_ref, dp_ref, b1_ref, w2_ref, att_ref,
              h2_ref, al_ref):
    den = jnp.sum(dp_ref[...], axis=0, keepdims=True) + 1e-16
    u = jnp.concatenate([ua_ref[...], ub_ref[...]], axis=1)
    g = jnp.maximum(u / den.T + b1_ref[...], 0.0)
    h2 = jnp.dot(g, w2_ref[...], preferred_element_type=jnp.float32)
    h2_ref[...] = h2
    a_src = jnp.sum(h2 * att_ref[0:1, :], axis=1, keepdims=True)
    a_dst = jnp.sum(h2 * att_ref[1:2, :], axis=1, keepdims=True)
    al_ref[...] = jnp.concatenate([a_src, a_dst], axis=1)


def _tc2(uA, uB, dp, bias1, W2, att2):
    return pl.pallas_call(
        _tc2_body,
        out_shape=[
            jax.ShapeDtypeStruct((N, H2), jnp.float32),
            jax.ShapeDtypeStruct((N, 2), jnp.float32),
        ],
    )(uA, uB, dp, bias1, W2, att2)


def _tc3_body(u0_ref, u1_ref, dp_ref, b2_ref, hf_ref):
    den = jnp.sum(dp_ref[...], axis=0, keepdims=True) + 1e-16
    u = u0_ref[...] + u1_ref[...]
    hf_ref[...] = u / den.T + b2_ref[...]


def _tc3(u0, u1, dp, bias2):
    return pl.pallas_call(
        _tc3_body,
        out_shape=jax.ShapeDtypeStruct((N, H2), jnp.float32),
    )(u0, u1, dp, bias2)


def _make_sc_gat(half):
    mesh = plsc.VectorSubcoreMesh(core_axis_name="c", subcore_axis_name="s",
                                  num_cores=NC, num_subcores=NS)

    @functools.partial(
        pl.kernel,
        out_type=[
            jax.ShapeDtypeStruct((NP, half), jnp.float32),
            jax.ShapeDtypeStruct((NP, half), jnp.float32),
            jax.ShapeDtypeStruct((NS, NP), jnp.float32),
        ],
        mesh=mesh,
        scratch_types=[
            pltpu.VMEM((2 * N,), jnp.float32),
            pltpu.VMEM((NP,), jnp.float32),
            pltpu.VMEM((B,), jnp.int32),
            pltpu.VMEM((B,), jnp.int32),
            pltpu.VMEM((B,), jnp.float32),
            pltpu.VMEM((B, half), jnp.float32),
            pltpu.VMEM_SHARED((NP, half), jnp.float32),
            pltpu.SemaphoreType.DMA,
        ],
        compiler_params=pltpu.CompilerParams(needs_layout_passes=False),
    )
    def k(ha_hbm, hb_hbm, al_hbm, src_hbm, dst_hbm,
          ua_out, ub_out, dp_out,
          al_v, dtab_v, src_v, dst_v, ex_v, rows_v, u_sh, sem):
        c = lax.axis_index("c")
        s = lax.axis_index("s")

        def zrow(i, carry):
            for kk in range(half // L):
                rows_v[i, pl.ds(kk * L, L)] = jnp.zeros((L,), jnp.float32)
            return carry

        lax.fori_loop(0, B, zrow, 0)
        for j in range(NPT // B):
            pltpu.sync_copy(rows_v,
                            u_sh.at[pl.ds(s * NPT + j * B, B)])

        def zden(i, carry):
            dtab_v[pl.ds(i * L, L)] = jnp.zeros((L,), jnp.float32)
            return carry

        lax.fori_loop(0, NP // L, zden, 0)
        pltpu.sync_copy(al_hbm, al_v)
        plsc.subcore_barrier()

        def batch(b, carry):
            base = s * EPT_MP + b * B
            pltpu.sync_copy(src_hbm.at[pl.ds(base, B)], src_v)
            pltpu.sync_copy(dst_hbm.at[pl.ds(base, B)], dst_v)

            @pl.when(c == 0)
            def _():
                pltpu.async_copy(ha_hbm.at[src_v], rows_v, sem).wait()

            @pl.when(c == 1)
            def _():
                pltpu.async_copy(hb_hbm.at[src_v], rows_v, sem).wait()

            def chunk(j, carry2):
                s16 = src_v[pl.ds(j * L, L)]
                d16 = dst_v[pl.ds(j * L, L)]
                av = plsc.load_gather(al_v, [s16 * 2])
                bv = plsc.load_gather(al_v, [d16 * 2 + 1])
                a = av + bv
                a = jnp.where(a > 0.0, a, 0.2 * a)
                ex = jnp.exp(a)
                pos = base + j * L + lax.iota(jnp.int32, L)
                ex = jnp.where(pos < E_MP, ex, 0.0)
                ex_v[pl.ds(j * L, L)] = ex

                @pl.when(c == 0)
                def _():
                    plsc.addupdate_scatter(dtab_v, [d16], ex)

                return carry2

            lax.fori_loop(0, B // L, chunk, 0)

            def scale(e, carry2):
                coef = plsc.load_gather(ex_v, [jnp.full((L,), e, jnp.int32)])
                for kk in range(half // L):
                    rows_v[e, pl.ds(kk * L, L)] = (
                        rows_v[e, pl.ds(kk * L, L)] * coef)
                return carry2

            lax.fori_loop(0, B, scale, 0)
            pltpu.sync_copy(rows_v, u_sh.at[dst_v], add=True)
            return carry

        lax.fori_loop(0, NB_MP, batch, 0)
        plsc.subcore_barrier()

        @pl.when(c == 0)
        def _():
            pltpu.sync_copy(u_sh.at[pl.ds(s * NPT, NPT)],
                            ua_out.at[pl.ds(s * NPT, NPT)])
            pltpu.sync_copy(dtab_v, dp_out.at[s])

        @pl.when(c == 1)
        def _():
            pltpu.sync_copy(u_sh.at[pl.ds(s * NPT, NPT)],
                            ub_out.at[pl.ds(s * NPT, NPT)])

    return k


NB_MP2 = E_MP_PAD // (NC * NS * B)
EPT_MP2 = NB_MP2 * B


def _make_sc_gat2():
    mesh = plsc.VectorSubcoreMesh(core_axis_name="c", subcore_axis_name="s",
                                  num_cores=NC, num_subcores=NS)

    @functools.partial(
        pl.kernel,
        out_type=[
            jax.ShapeDtypeStruct((NC, NP, H2), jnp.float32),
            jax.ShapeDtypeStruct((NC, NS, NP), jnp.float32),
        ],
        mesh=mesh,
        scratch_types=[
            pltpu.VMEM((2 * N,), jnp.float32),
            pltpu.VMEM((NP,), jnp.float32),
            pltpu.VMEM((B,), jnp.int32),
            pltpu.VMEM((B,), jnp.int32),
            pltpu.VMEM((B,), jnp.float32),
            pltpu.VMEM((B, H2), jnp.float32),
            pltpu.VMEM_SHARED((NP, H2), jnp.float32),
            pltpu.SemaphoreType.DMA,
        ],
        compiler_params=pltpu.CompilerParams(needs_layout_passes=False),
    )
    def k(h_hbm, al_hbm, src_hbm, dst_hbm,
          u_out, dp_out,
          al_v, dtab_v, src_v, dst_v, ex_v, rows_v, u_sh, sem):
        c = lax.axis_index("c")
        s = lax.axis_index("s")
        w = c * NS + s

        def zrow(i, carry):
            for kk in range(H2 // L):
                rows_v[i, pl.ds(kk * L, L)] = jnp.zeros((L,), jnp.float32)
            return carry

        lax.fori_loop(0, B, zrow, 0)
        for j in range(NPT // B):
            pltpu.sync_copy(rows_v,
                            u_sh.at[pl.ds(s * NPT + j * B, B)])

        def zden(i, carry):
            dtab_v[pl.ds(i * L, L)] = jnp.zeros((L,), jnp.float32)
            return carry

        lax.fori_loop(0, NP // L, zden, 0)
        pltpu.sync_copy(al_hbm, al_v)
        plsc.subcore_barrier()

        def batch(b, carry):
            base = w * EPT_MP2 + b * B
            pltpu.sync_copy(src_hbm.at[pl.ds(base, B)], src_v)
            pltpu.sync_copy(dst_hbm.at[pl.ds(base, B)], dst_v)
            pltpu.async_copy(h_hbm.at[src_v], rows_v, sem).wait()

            def chunk(j, carry2):
                s16 = src_v[pl.ds(j * L, L)]
                d16 = dst_v[pl.ds(j * L, L)]
                av = plsc.load_gather(al_v, [s16 * 2])
                bv = plsc.load_gather(al_v, [d16 * 2 + 1])
                a = av + bv
                a = jnp.where(a > 0.0, a, 0.2 * a)
                ex = jnp.exp(a)
                pos = base + j * L + lax.iota(jnp.int32, L)
                ex = jnp.where(pos < E_MP, ex, 0.0)
                ex_v[pl.ds(j * L, L)] = ex
                plsc.addupdate_scatter(dtab_v, [d16], ex)
                return carry2

            lax.fori_loop(0, B // L, chunk, 0)

            def scale(e, carry2):
                coef = plsc.load_gather(ex_v, [jnp.full((L,), e, jnp.int32)])
                for kk in range(H2 // L):
                    rows_v[e, pl.ds(kk * L, L)] = (
                        rows_v[e, pl.ds(kk * L, L)] * coef)
                return carry2

            lax.fori_loop(0, B, scale, 0)
            pltpu.sync_copy(rows_v, u_sh.at[dst_v], add=True)
            return carry

        lax.fori_loop(0, NB_MP2, batch, 0)
        plsc.subcore_barrier()
        pltpu.sync_copy(u_sh.at[pl.ds(s * NPT, NPT)],
                        u_out.at[c, pl.ds(s * NPT, NPT)])
        pltpu.sync_copy(dtab_v, dp_out.at[c, s])

    return k


def _make_sc_head():
    mesh = plsc.VectorSubcoreMesh(core_axis_name="c", subcore_axis_name="s",
                                  num_cores=NC, num_subcores=NS)

    @functools.partial(
        pl.kernel,
        out_type=jax.ShapeDtypeStruct((E_HD_PAD,), jnp.float32),
        mesh=mesh,
        scratch_types=[
            pltpu.VMEM((B,), jnp.int32),
            pltpu.VMEM((B,), jnp.int32),
            pltpu.VMEM((B, H2), jnp.float32),
            pltpu.VMEM((B, H2), jnp.float32),
            pltpu.VMEM((B,), jnp.float32),
            pltpu.SemaphoreType.DMA,
        ],
        compiler_params=pltpu.CompilerParams(needs_layout_passes=False),
    )
    def k(hf_hbm, e0_hbm, e1_hbm, out_hbm,
          e0_v, e1_v, r0_v, r1_v, ob_v, sem):
        c = lax.axis_index("c")
        s = lax.axis_index("s")
        w = c * NS + s

        def batch(b, carry):
            base = (w * NB_HD + b) * B
            pltpu.sync_copy(e0_hbm.at[pl.ds(base, B)], e0_v)
            pltpu.sync_copy(e1_hbm.at[pl.ds(base, B)], e1_v)
            pltpu.async_copy(hf_hbm.at[e0_v], r0_v, sem).wait()
            pltpu.async_copy(hf_hbm.at[e1_v], r1_v, sem).wait()

            lane0 = lax.iota(jnp.int32, L) == 0

            def dot_e(e, carry2):
                acc = r0_v[e, pl.ds(0, L)] * r1_v[e, pl.ds(0, L)]
                for kk in range(1, H2 // L):
                    acc = acc + (r0_v[e, pl.ds(kk * L, L)]
                                 * r1_v[e, pl.ds(kk * L, L)])
                s = jnp.sum(acc)
                plsc.store_scatter(ob_v, [jnp.full((L,), e, jnp.int32)],
                                   jnp.full((L,), s), mask=lane0)
                return carry2

            lax.fori_loop(0, B, dot_e, 0)
            for kk in range(B // L):
                v = ob_v[pl.ds(kk * L, L)]
                ob_v[pl.ds(kk * L, L)] = 1.0 / (1.0 + jnp.exp(-v))
            pltpu.sync_copy(ob_v, out_hbm.at[pl.ds(base, B)])
            return carry

        lax.fori_loop(0, NB_HD, batch, 0)

    return k


def kernel(x, edge_index, edge, W0, b0, W1, att_src1, att_dst1, bias1,
           W2, att_src2, att_dst2, bias2):
    loop = jnp.arange(N, dtype=jnp.int32)
    src = jnp.concatenate([edge_index[0].astype(jnp.int32), loop])
    dst = jnp.concatenate([edge_index[1].astype(jnp.int32), loop])
    src = jnp.pad(src, (0, E_MP_PAD - E_MP))
    dst = jnp.pad(dst, (0, E_MP_PAD - E_MP))
    e0 = jnp.pad(edge[0].astype(jnp.int32), (0, E_HD_PAD - E))
    e1 = jnp.pad(edge[1].astype(jnp.int32), (0, E_HD_PAD - E))

    att1 = jnp.stack([att_src1, att_dst1])
    att2 = jnp.stack([att_src2, att_dst2])

    hA, hB, al1 = _tc1(x, W0, b0.reshape(1, -1), W1, att1)
    uA, uB, dp1 = _make_sc_gat(H1 // 2)(hA, hB, al1.reshape(-1), src, dst)
    h2, al2 = _tc2(uA[:N], uB[:N], dp1[:, :N],
                   bias1.reshape(1, -1), W2, att2)
    u2, dp2 = _make_sc_gat2()(h2, al2.reshape(-1), src, dst)
    hf = _tc3(u2[0, :N], u2[1, :N], dp2.reshape(NC * NS, NP)[:, :N],
              bias2.reshape(1, -1))
    score = _make_sc_head()(hf, e0, e1)
    return score[:E]
